# baseline (speedup 1.0000x reference)
import functools

import jax
import jax.numpy as jnp
from jax import lax
from jax.experimental import pallas as pl
from jax.experimental.pallas import tpu as pltpu
from jax.experimental.pallas import tpu_sc as plsc

BATCH = 4
SEQ = 8192
D = 1024
LANES = 16

R = 4
NBUF = 4
NW = 32
ROWS_PER_W = SEQ // NW
N_CHUNKS = ROWS_PER_W // R


def _sc_body(x_hbm, pos_hbm, out_hbm, x_v, pos_v, *sems):
    sin = sems[:NBUF]
    sout = sems[NBUF:]
    wid = lax.axis_index("s") * 2 + lax.axis_index("c")
    s0 = wid * ROWS_PER_W

    def start_in(c, k):
        row = s0 + c * R
        pltpu.make_async_copy(pos_hbm.at[pl.ds(row, R), :], pos_v.at[k], sin[k]).start()
        pltpu.make_async_copy(x_hbm.at[:, pl.ds(row, R), :], x_v.at[k], sin[k]).start()

    def wait_in(k):
        pltpu.make_async_copy(pos_hbm.at[pl.ds(0, R), :], pos_v.at[k], sin[k]).wait()
        pltpu.make_async_copy(x_hbm.at[:, pl.ds(0, R), :], x_v.at[k], sin[k]).wait()

    def start_out(c, k):
        row = s0 + c * R
        pltpu.make_async_copy(x_v.at[k], out_hbm.at[:, pl.ds(row, R), :], sout[k]).start()

    def wait_out(k):
        pltpu.make_async_copy(x_v.at[k], out_hbm.at[:, pl.ds(0, R), :], sout[k]).wait()

    def compute(k):
        @plsc.parallel_loop(0, D // LANES, carry=jnp.int32(0))
        def col(i, carry):
            cs = i * LANES
            for r in range(R):
                p = pos_v[k, r, pl.ds(cs, LANES)]
                for b in range(BATCH):
                    x_v[k, b, r, pl.ds(cs, LANES)] = x_v[k, b, r, pl.ds(cs, LANES)] + p
            return carry

    start_in(0, 0)
    start_in(1, 1)

    def quad(cc, carry):
        for kk in range(NBUF):
            c = cc * NBUF + kk

            @pl.when(c >= 2)
            def _():
                wait_out((kk + 2) % NBUF)

            @pl.when(c + 2 < N_CHUNKS)
            def _():
                start_in(c + 2, (kk + 2) % NBUF)

            wait_in(kk)
            compute(kk)
            start_out(c, kk)
        return carry

    lax.fori_loop(0, N_CHUNKS // NBUF, quad, 0)
    wait_out((N_CHUNKS - 2) % NBUF)
    wait_out((N_CHUNKS - 1) % NBUF)


def kernel(x, pos_embedding):
    batch, seq_len, d_model = x.shape
    mesh = plsc.VectorSubcoreMesh(core_axis_name="c", subcore_axis_name="s")
    f = functools.partial(
        pl.kernel,
        out_type=jax.ShapeDtypeStruct((batch, seq_len, d_model), x.dtype),
        mesh=mesh,
        scratch_types=[
            pltpu.VMEM((NBUF, BATCH, R, D), jnp.float32),
            pltpu.VMEM((NBUF, R, D), jnp.float32),
        ]
        + [pltpu.SemaphoreType.DMA] * (2 * NBUF),
    )(_sc_body)
    return f(x, pos_embedding[:seq_len])


# speedup vs baseline: 1.0478x; 1.0478x over previous
import functools

import jax
import jax.numpy as jnp
from jax import lax
from jax.experimental import pallas as pl
from jax.experimental.pallas import tpu as pltpu
from jax.experimental.pallas import tpu_sc as plsc

BATCH = 4
SEQ = 8192
D = 1024
LANES = 16

SC_ROWS = 2048
TC_ROWS = SEQ - SC_ROWS

R = 4
NBUF = 4
NW = 32
ROWS_PER_W = SC_ROWS // NW
N_CHUNKS = ROWS_PER_W // R

S_BLK = 512


def _sc_body(x_hbm, pos_hbm, out_hbm, x_v, pos_v, *sems):
    sin = sems[:NBUF]
    sout = sems[NBUF:]
    wid = lax.axis_index("s") * 2 + lax.axis_index("c")
    s0 = TC_ROWS + wid * ROWS_PER_W

    def start_in(c, k):
        row = s0 + c * R
        pltpu.make_async_copy(pos_hbm.at[pl.ds(row, R), :], pos_v.at[k], sin[k]).start()
        pltpu.make_async_copy(x_hbm.at[:, pl.ds(row, R), :], x_v.at[k], sin[k]).start()

    def wait_in(k):
        pltpu.make_async_copy(pos_hbm.at[pl.ds(0, R), :], pos_v.at[k], sin[k]).wait()
        pltpu.make_async_copy(x_hbm.at[:, pl.ds(0, R), :], x_v.at[k], sin[k]).wait()

    def start_out(c, k):
        row = s0 + c * R
        pltpu.make_async_copy(x_v.at[k], out_hbm.at[:, pl.ds(row, R), :], sout[k]).start()

    def wait_out(k):
        pltpu.make_async_copy(x_v.at[k], out_hbm.at[:, pl.ds(0, R), :], sout[k]).wait()

    def compute(k):
        @plsc.parallel_loop(0, D // LANES, carry=jnp.int32(0))
        def col(i, carry):
            cs = i * LANES
            for r in range(R):
                p = pos_v[k, r, pl.ds(cs, LANES)]
                for b in range(BATCH):
                    x_v[k, b, r, pl.ds(cs, LANES)] = x_v[k, b, r, pl.ds(cs, LANES)] + p
            return carry

    start_in(0, 0)
    start_in(1, 1)

    def quad(cc, carry):
        for kk in range(NBUF):
            c = cc * NBUF + kk

            @pl.when(c >= 2)
            def _():
                wait_out((kk + 2) % NBUF)

            @pl.when(c + 2 < N_CHUNKS)
            def _():
                start_in(c + 2, (kk + 2) % NBUF)

            wait_in(kk)
            compute(kk)
            start_out(c, kk)
        return carry

    lax.fori_loop(0, N_CHUNKS // NBUF, quad, 0)
    wait_out((N_CHUNKS - 2) % NBUF)
    wait_out((N_CHUNKS - 1) % NBUF)


def _tc_body(buf_ref, x_ref, pos_ref, out_ref):
    out_ref[...] = x_ref[...] + pos_ref[...][None, :, :]


def kernel(x, pos_embedding):
    batch, seq_len, d_model = x.shape
    pos = pos_embedding[:seq_len]

    mesh = plsc.VectorSubcoreMesh(core_axis_name="c", subcore_axis_name="s")
    sc_out = functools.partial(
        pl.kernel,
        out_type=jax.ShapeDtypeStruct((batch, seq_len, d_model), x.dtype),
        mesh=mesh,
        scratch_types=[
            pltpu.VMEM((NBUF, BATCH, R, D), jnp.float32),
            pltpu.VMEM((NBUF, R, D), jnp.float32),
        ]
        + [pltpu.SemaphoreType.DMA] * (2 * NBUF),
    )(_sc_body)(x, pos)

    n_tc = TC_ROWS // S_BLK
    return pl.pallas_call(
        _tc_body,
        grid=(n_tc,),
        in_specs=[
            pl.BlockSpec(memory_space=pl.ANY),
            pl.BlockSpec((batch, S_BLK, d_model), lambda s: (0, s, 0)),
            pl.BlockSpec((S_BLK, d_model), lambda s: (s, 0)),
        ],
        out_specs=pl.BlockSpec((batch, S_BLK, d_model), lambda s: (0, s, 0)),
        out_shape=jax.ShapeDtypeStruct((batch, seq_len, d_model), x.dtype),
        input_output_aliases={0: 0},
    )(sc_out, x, pos)


# speedup vs baseline: 1.0566x; 1.0084x over previous
import functools

import jax
import jax.numpy as jnp
from jax import lax
from jax.experimental import pallas as pl
from jax.experimental.pallas import tpu as pltpu
from jax.experimental.pallas import tpu_sc as plsc

BATCH = 4
SEQ = 8192
D = 1024
LANES = 16

SC_ROWS = 1024
TC_ROWS = SEQ - SC_ROWS

R = 4
NBUF = 4
NW = 32
ROWS_PER_W = SC_ROWS // NW
N_CHUNKS = ROWS_PER_W // R

S_BLK = 512


def _sc_body(x_hbm, pos_hbm, out_hbm, x_v, pos_v, *sems):
    sin = sems[:NBUF]
    sout = sems[NBUF:]
    wid = lax.axis_index("s") * 2 + lax.axis_index("c")
    s0 = TC_ROWS + wid * ROWS_PER_W

    def start_in(c, k):
        row = s0 + c * R
        pltpu.make_async_copy(pos_hbm.at[pl.ds(row, R), :], pos_v.at[k], sin[k]).start()
        pltpu.make_async_copy(x_hbm.at[:, pl.ds(row, R), :], x_v.at[k], sin[k]).start()

    def wait_in(k):
        pltpu.make_async_copy(pos_hbm.at[pl.ds(0, R), :], pos_v.at[k], sin[k]).wait()
        pltpu.make_async_copy(x_hbm.at[:, pl.ds(0, R), :], x_v.at[k], sin[k]).wait()

    def start_out(c, k):
        row = s0 + c * R
        pltpu.make_async_copy(x_v.at[k], out_hbm.at[:, pl.ds(row, R), :], sout[k]).start()

    def wait_out(k):
        pltpu.make_async_copy(x_v.at[k], out_hbm.at[:, pl.ds(0, R), :], sout[k]).wait()

    def compute(k):
        @plsc.parallel_loop(0, D // LANES, carry=jnp.int32(0))
        def col(i, carry):
            cs = i * LANES
            for r in range(R):
                p = pos_v[k, r, pl.ds(cs, LANES)]
                for b in range(BATCH):
                    x_v[k, b, r, pl.ds(cs, LANES)] = x_v[k, b, r, pl.ds(cs, LANES)] + p
            return carry

    start_in(0, 0)
    start_in(1, 1)

    def quad(cc, carry):
        for kk in range(NBUF):
            c = cc * NBUF + kk

            @pl.when(c >= 2)
            def _():
                wait_out((kk + 2) % NBUF)

            @pl.when(c + 2 < N_CHUNKS)
            def _():
                start_in(c + 2, (kk + 2) % NBUF)

            wait_in(kk)
            compute(kk)
            start_out(c, kk)
        return carry

    lax.fori_loop(0, N_CHUNKS // NBUF, quad, 0)
    wait_out((N_CHUNKS - 2) % NBUF)
    wait_out((N_CHUNKS - 1) % NBUF)


def _tc_body(buf_ref, x_ref, pos_ref, out_ref):
    out_ref[...] = x_ref[...] + pos_ref[...][None, :, :]


def kernel(x, pos_embedding):
    batch, seq_len, d_model = x.shape
    pos = pos_embedding[:seq_len]

    mesh = plsc.VectorSubcoreMesh(core_axis_name="c", subcore_axis_name="s")
    sc_out = functools.partial(
        pl.kernel,
        out_type=jax.ShapeDtypeStruct((batch, seq_len, d_model), x.dtype),
        mesh=mesh,
        scratch_types=[
            pltpu.VMEM((NBUF, BATCH, R, D), jnp.float32),
            pltpu.VMEM((NBUF, R, D), jnp.float32),
        ]
        + [pltpu.SemaphoreType.DMA] * (2 * NBUF),
    )(_sc_body)(x, pos)

    n_tc = TC_ROWS // S_BLK
    return pl.pallas_call(
        _tc_body,
        grid=(n_tc,),
        in_specs=[
            pl.BlockSpec(memory_space=pl.ANY),
            pl.BlockSpec((batch, S_BLK, d_model), lambda s: (0, s, 0)),
            pl.BlockSpec((S_BLK, d_model), lambda s: (s, 0)),
        ],
        out_specs=pl.BlockSpec((batch, S_BLK, d_model), lambda s: (0, s, 0)),
        out_shape=jax.ShapeDtypeStruct((batch, seq_len, d_model), x.dtype),
        input_output_aliases={0: 0},
    )(sc_out, x, pos)
